# baseline (speedup 1.0000x reference)
import functools

import jax
import jax.numpy as jnp
from jax import lax
from jax.experimental import pallas as pl
from jax.experimental.pallas import tpu as pltpu
from jax.experimental.pallas import tpu_sc as plsc

N_ITEM = 1000000
N_DIM = 64
BATCH = 16384

NC = 2
NS = 16
L = 16
NW = NC * NS
SPAN = BATCH // NS
SLOT_W = 4096
HALF_W = SLOT_W // 2
N_SLOTS = 245
LAST0 = (N_SLOTS - 1) * SLOT_W
NK2 = 62
RING = 8


def _p1_body(idx_hbm, tabT, inter_hbm,
             idxv, ring, sp0, sp1, tail,
             csr_s, cnt_s, off_s, ssem, xsem, wsem):
    cid = lax.axis_index("c")
    sid = lax.axis_index("s")
    span0 = sid * SPAN

    pltpu.sync_copy(idx_hbm.at[pl.ds(span0, SPAN)], idxv.at[pl.ds(0, SPAN)])

    bufs = [sp0, sp1]
    my_tr = sid % 8
    my_half = sid // 8

    def fire_slab(s, b):
        @pl.when(s < N_SLOTS - 1)
        def _():
            for h in range(2):
                @pl.when(my_half == h)
                def _():
                    pltpu.async_copy(
                        tabT.at[pl.ds(8 * my_tr, 8),
                                pl.ds(s * SLOT_W + h * HALF_W, HALF_W)],
                        bufs[b].at[my_tr, :, pl.ds(h * HALF_W, HALF_W)],
                        ssem)

        @pl.when(s == N_SLOTS - 1)
        def _():
            for h in range(2):
                @pl.when(my_half == h)
                def _():
                    pltpu.async_copy(
                        tabT.at[pl.ds(8 * my_tr, 8),
                                pl.ds(LAST0 + h * 256, 256)],
                        bufs[b].at[my_tr, :, pl.ds(h * 256, 256)],
                        ssem)

    def wait_slab(s, b):
        @pl.when(s < N_SLOTS - 1)
        def _():
            for h in range(2):
                @pl.when(my_half == h)
                def _():
                    pltpu.make_async_copy(
                        tabT.at[pl.ds(0, 8), pl.ds(0, HALF_W)],
                        bufs[b].at[0, :, pl.ds(0, HALF_W)], ssem).wait()

        @pl.when(s == N_SLOTS - 1)
        def _():
            for h in range(2):
                @pl.when(my_half == h)
                def _():
                    pltpu.make_async_copy(
                        tabT.at[pl.ds(0, 8), pl.ds(0, 256)],
                        bufs[b].at[0, :, pl.ds(0, 256)], ssem).wait()

    def zero(i, _):
        cnt_s[i] = 0
        return 0
    lax.fori_loop(0, N_SLOTS, zero, 0)

    def read_idx(pos):
        return idxv[pl.ds(pos, L)][0]

    def count(pos, _):
        s = read_idx(pos) >> 12
        @pl.when((s & 1) == cid)
        def _():
            cnt_s[s] = cnt_s[s] + 1
        return 0
    lax.fori_loop(0, SPAN, count, 0)

    def prefix(s, carry):
        off_s[s] = carry
        return carry + cnt_s[s]
    total = lax.fori_loop(0, N_SLOTS, prefix, 0)
    off_s[N_SLOTS] = total

    def cursor(s, _):
        cnt_s[s] = off_s[s]
        return 0
    lax.fori_loop(0, N_SLOTS, cursor, 0)

    def place(pos, _):
        s = read_idx(pos) >> 12
        @pl.when((s & 1) == cid)
        def _():
            e = cnt_s[s]
            cnt_s[s] = e + 1
            w = csr_s[e >> 1]
            sh = (e & 1) * 16
            keep = jnp.bitwise_not(jnp.int32(0xFFFF) << sh)
            csr_s[e >> 1] = (w & keep) | ((pos & jnp.int32(0xFFFF)) << sh)
        return 0
    lax.fori_loop(0, SPAN, place, 0)

    @pl.when(my_half == 0)
    def _():
        pltpu.async_copy(
            tabT.at[pl.ds(8 * my_tr, 8), pl.ds(LAST0 + 512, 64)],
            tail.at[my_tr, :, pl.ds(0, 64)], ssem).wait()

    fire_slab(cid, 0)

    def entry(i):
        w = csr_s[i >> 1]
        pos = (w >> ((i & 1) * 16)) & jnp.int32(0xFFFF)
        idx = read_idx(pos)
        p = span0 + pos
        off = idx & (SLOT_W - 1)
        return p, off

    zq0 = jnp.minimum(cid, 0)

    def drain_x():
        pltpu.make_async_copy(sp0.at[:, :, pl.ds(0, 1)],
                              ring.at[:, :, pl.ds(0, 1)], xsem).wait()

    def drain_w():
        pltpu.make_async_copy(ring.at[:, :, pl.ds(0, 1)],
                              inter_hbm.at[0, :, :, pl.ds(zq0, 1)],
                              wsem).wait()

    def do_slot(k, b):
        s_cur = 2 * k + cid
        wait_slab(s_cur, b)
        plsc.subcore_barrier()

        @pl.when(k + 1 < 2 * NK2)
        def _():
            fire_slab(2 * (k + 1) + cid, 1 - b)

        s_ok = s_cur < N_SLOTS
        s_eff = jnp.minimum(s_cur, N_SLOTS - 1)
        lo = off_s[s_eff]
        hi = off_s[s_eff + 1]
        n = jnp.where(s_ok, hi - lo, 0)
        is_last = s_cur == N_SLOTS - 1

        def fire_extract(off, slot):
            in_tail = jnp.logical_and(is_last, off >= jnp.int32(512))

            @pl.when(jnp.logical_not(in_tail))
            def _():
                pltpu.async_copy(bufs[b].at[:, :, pl.ds(off, 1)],
                                 ring.at[:, :, pl.ds(slot, 1)], xsem)

            @pl.when(in_tail)
            def _():
                pltpu.async_copy(tail.at[:, :, pl.ds(off - 512, 1)],
                                 ring.at[:, :, pl.ds(slot, 1)], xsem)

        def wr(j):
            p, _o = entry(lo + j)
            drain_x()
            pltpu.async_copy(ring.at[:, :, pl.ds(j & (RING - 1), 1)],
                             inter_hbm.at[p, :, :,
                                          pl.ds(jnp.minimum(p, 0), 1)], wsem)

        def istep(i, _):
            p, off = entry(lo + i)
            slot = i & (RING - 1)

            @pl.when(i >= RING)
            def _():
                drain_w()
            fire_extract(off, slot)

            @pl.when(i >= 1)
            def _():
                wr(i - 1)
            return 0
        lax.fori_loop(0, n, istep, 0)

        @pl.when(n >= 1)
        def _():
            wr(n - 1)

        def wdrain(i, _):
            drain_w()
            return 0
        lax.fori_loop(0, jnp.minimum(n, RING), wdrain, 0)
        plsc.subcore_barrier()

    def kloop2(kk, _):
        do_slot(2 * kk, 0)
        do_slot(2 * kk + 1, 1)
        return 0

    lax.fori_loop(0, NK2, kloop2, 0)


def _mesh():
    return plsc.VectorSubcoreMesh(core_axis_name="c", subcore_axis_name="s",
                                  num_cores=NC, num_subcores=NS)


def _p1(idx, tabT):
    return pl.kernel(
        _p1_body,
        out_type=jax.ShapeDtypeStruct((BATCH, 8, 8, 1), jnp.float32),
        mesh=_mesh(),
        compiler_params=pltpu.CompilerParams(needs_layout_passes=False),
        scratch_types=[
            pltpu.VMEM((SPAN + L,), jnp.int32),
            pltpu.VMEM((8, 8, RING), jnp.float32),
            pltpu.VMEM_SHARED((8, 8, SLOT_W), jnp.float32),
            pltpu.VMEM_SHARED((8, 8, SLOT_W), jnp.float32),
            pltpu.VMEM_SHARED((8, 8, 64), jnp.float32),
            pltpu.SMEM((SPAN // 2 + 8,), jnp.int32),
            pltpu.SMEM((N_SLOTS + 8,), jnp.int32),
            pltpu.SMEM((N_SLOTS + 8,), jnp.int32),
            pltpu.SemaphoreType.DMA,
            pltpu.SemaphoreType.DMA,
            pltpu.SemaphoreType.DMA,
        ],
    )(idx, tabT)


def _tc_dot_body(w_ref, c_ref, o_ref):
    z = jnp.sum(w_ref[...] * c_ref[...], axis=1)
    o_ref[...] = jax.nn.sigmoid(z)


@jax.jit
def skipgram_v4(word, ctx, wtabT, ctabT):
    iw = _p1(word, wtabT).reshape(BATCH, N_DIM)
    ic = _p1(ctx, ctabT).reshape(BATCH, N_DIM)
    return pl.pallas_call(
        _tc_dot_body,
        out_shape=jax.ShapeDtypeStruct((BATCH,), jnp.float32),
        grid=(16,),
        in_specs=[
            pl.BlockSpec((BATCH // 16, N_DIM), lambda i: (i, 0)),
            pl.BlockSpec((BATCH // 16, N_DIM), lambda i: (i, 0)),
        ],
        out_specs=pl.BlockSpec((BATCH // 16,), lambda i: (i,)),
    )(iw, ic)


def kernel_v4(word, context, word_embeddings, context_embeddings):
    return skipgram_v4(word.astype(jnp.int32), context.astype(jnp.int32),
                       word_embeddings.T, context_embeddings.T)


def kernel(word, context, word_embeddings, context_embeddings):
    return kernel_v4(word, context, word_embeddings, context_embeddings)


# speedup vs baseline: 2.1882x; 2.1882x over previous
import functools

import jax
import jax.numpy as jnp
from jax import lax
from jax.experimental import pallas as pl
from jax.experimental.pallas import tpu as pltpu
from jax.experimental.pallas import tpu_sc as plsc

N_ITEM = 1000000
N_DIM = 64
BATCH = 16384

NC = 2
NS = 16
L = 16
NW = NC * NS
B_PER_W = BATCH // NW
CHUNK = 128
N_CHUNKS = B_PER_W // CHUNK
PAIR_W = 2 * N_DIM


def _sc_body(word_hbm, ctx_hbm, wp_hbm, cp_hbm, wtab_hbm, ctab_hbm, out_hbm,
             widx, cidx, wpair, cpair, wr0, wr1, cr0, cr1, out_v, sems):
    wid = lax.axis_index("s") * NC + lax.axis_index("c")
    base = wid * B_PER_W

    pltpu.sync_copy(word_hbm.at[pl.ds(base, B_PER_W)], widx)
    pltpu.sync_copy(ctx_hbm.at[pl.ds(base, B_PER_W)], cidx)
    pltpu.sync_copy(wp_hbm.at[pl.ds(base, B_PER_W)], wpair)
    pltpu.sync_copy(cp_hbm.at[pl.ds(base, B_PER_W)], cpair)

    wbufs = [wr0, wr1]
    cbufs = [cr0, cr1]

    def fire(c):
        return (
            pltpu.async_copy(
                wtab_hbm.at[wpair.at[pl.ds(c * CHUNK, CHUNK)]],
                wbufs[c % 2], sems.at[2 * (c % 2)]),
            pltpu.async_copy(
                ctab_hbm.at[cpair.at[pl.ds(c * CHUNK, CHUNK)]],
                cbufs[c % 2], sems.at[2 * (c % 2) + 1]),
        )

    pending = {0: fire(0), 1: fire(1)}

    for c in range(N_CHUNKS):
        pending[c][0].wait()
        pending[c][1].wait()
        wbuf, cbuf = wbufs[c % 2], cbufs[c % 2]

        def body(g, _, c=c, wbuf=wbuf, cbuf=cbuf):
            gbase = c * CHUNK + g * L
            ridx = jnp.arange(L, dtype=jnp.int32) + g * L
            wcol0 = (widx[pl.ds(gbase, L)] & 1) * N_DIM
            ccol0 = (cidx[pl.ds(gbase, L)] & 1) * N_DIM
            acc = jnp.zeros((L,), jnp.float32)
            for j in range(N_DIM):
                w = plsc.load_gather(wbuf, [ridx, wcol0 + j])
                x = plsc.load_gather(cbuf, [ridx, ccol0 + j])
                acc = acc + w * x
            out_v[pl.ds(gbase, L)] = 1.0 / (1.0 + jnp.exp(-acc))
            return 0

        lax.fori_loop(0, CHUNK // L, body, 0)
        if c + 2 < N_CHUNKS:
            pending[c + 2] = fire(c + 2)

    pltpu.sync_copy(out_v, out_hbm.at[pl.ds(base, B_PER_W)])


@jax.jit
def _skipgram_sc(word, ctx, wpair, cpair, wtab2, ctab2):
    mesh = plsc.VectorSubcoreMesh(core_axis_name="c", subcore_axis_name="s",
                                  num_cores=NC, num_subcores=NS)
    return pl.kernel(
        _sc_body,
        out_type=jax.ShapeDtypeStruct((BATCH,), jnp.float32),
        mesh=mesh,
        compiler_params=pltpu.CompilerParams(needs_layout_passes=False),
        scratch_types=[
            pltpu.VMEM((B_PER_W,), jnp.int32),
            pltpu.VMEM((B_PER_W,), jnp.int32),
            pltpu.VMEM((B_PER_W,), jnp.int32),
            pltpu.VMEM((B_PER_W,), jnp.int32),
            pltpu.VMEM((CHUNK, PAIR_W), jnp.float32),
            pltpu.VMEM((CHUNK, PAIR_W), jnp.float32),
            pltpu.VMEM((CHUNK, PAIR_W), jnp.float32),
            pltpu.VMEM((CHUNK, PAIR_W), jnp.float32),
            pltpu.VMEM((B_PER_W,), jnp.float32),
            pltpu.SemaphoreType.DMA((4,)),
        ],
    )(word, ctx, wpair, cpair, wtab2, ctab2)


def kernel(word, context, word_embeddings, context_embeddings):
    wtab2 = word_embeddings.reshape(N_ITEM // 2, PAIR_W)
    ctab2 = context_embeddings.reshape(N_ITEM // 2, PAIR_W)
    w32 = word.astype(jnp.int32)
    c32 = context.astype(jnp.int32)
    return _skipgram_sc(w32, c32, w32 >> 1, c32 >> 1, wtab2, ctab2)
